# baseline (speedup 1.0000x reference)
import jax
import jax.numpy as jnp
from jax import lax
from jax.experimental import pallas as pl
from jax.experimental.pallas import tpu as pltpu
from jax.experimental.pallas import tpu_sc as plsc

NVOC = 1000000
K = 32
BATCH = 16384

NC = 2
NS = 16
NW = NC * NS
W = 256
NWIN = 122
RNG = NWIN * W
NBUF = 2
TAIL0 = RNG * NW
TAIL1 = TAIL0 + 2 * W
L = 16


def _sc_body(u_hbm, i_hbm, t2u, t2i, ub_hbm, ib_hbm,
             ue_o, ie_o, ub_o, ib_o,
             idx_v, lu_v, lj_v, bufs, wq_u, wq_j, stag, bidx, bgi, bgv,
             sems, ssem, bsem):
    cid = lax.axis_index("c")
    sid = lax.axis_index("s")
    rid = cid * NS + sid
    w0 = rid * RNG
    iota16 = lax.iota(jnp.int32, L)

    def issue(tab, g, slot):
        for a in range(4):
            pltpu.async_copy(tab.at[pl.ds(a * 8, 8), pl.ds(w0 + g * W, W)],
                             bufs.at[slot, pl.ds(a * 8, 8), :], sems.at[slot])

    def wait_win(tab, g, slot):
        for a in range(4):
            pltpu.make_async_copy(
                tab.at[pl.ds(a * 8, 8), pl.ds(w0 + g * W, W)],
                bufs.at[slot, pl.ds(a * 8, 8), :], sems.at[slot]).wait()

    def drain_scatter(out_o, p):
        for q in range(4):
            pltpu.make_async_copy(stag.at[p, pl.ds(q * 128, 128)],
                                  out_o.at[bidx.at[p, q]],
                                  ssem.at[p]).wait()

    def one_table(tab, idx2d_hbm, bias_hbm, out_o, bias_o):
        def filt_half(h, pos):
            pltpu.sync_copy(idx2d_hbm.at[pl.ds(h * 64, 64), :], idx_v)

            def filt_row(r, pos):
                def sub(t, pos):
                    uv = idx_v[r, pl.ds(t * L, L)]
                    own = jnp.minimum(uv // RNG, NW - 1)
                    m = own == rid
                    jv = (h * 64 + r) * 128 + t * L + iota16
                    plsc.store_compressed(lu_v.at[pl.ds(pos, L)], uv, mask=m)
                    plsc.store_compressed(lj_v.at[pl.ds(pos, L)], jv, mask=m)
                    return pos + plsc.all_reduce_population_count(m)[0]
                for t in range(8):
                    pos = sub(t, pos)
                return pos
            return lax.fori_loop(0, 64, filt_row, pos)

        nloc = filt_half(0, jnp.int32(0))
        nloc = filt_half(1, nloc)

        def extract_window(v0, ww, slot, gcnt):
            def scan_chunk(c, nq):
                uv = lu_v[pl.ds(c * L, L)]
                jv = lj_v[pl.ds(c * L, L)]
                off = uv - v0
                m = (c * L + iota16 < nloc) & (off >= 0) & (off < ww)
                plsc.store_compressed(wq_u.at[pl.ds(nq, L)], uv, mask=m)
                plsc.store_compressed(wq_j.at[pl.ds(nq, L)], jv, mask=m)
                return nq + plsc.all_reduce_population_count(m)[0]
            nq = lax.fori_loop(0, (nloc + L - 1) // L, scan_chunk,
                               jnp.int32(0))

            def group(qb, gcnt):
                n = nq - qb * L
                p = gcnt % NBUF

                @pl.when(gcnt >= NBUF)
                def _():
                    drain_scatter(out_o, p)

                jlast = wq_j[pl.ds(qb * L + n - 1, L)][0]
                ulast = wq_u[pl.ds(qb * L + n - 1, L)][0]
                jq = wq_j[pl.ds(qb * L, L)]
                uq = wq_u[pl.ds(qb * L, L)]
                jqp = jnp.where(iota16 < n, jq, jlast)
                offs = jnp.where(iota16 < n, uq, ulast) - v0

                for e in range(L):
                    offv = iota16 * 0 + offs[e]
                    lo = plsc.load_gather(bufs.at[slot], [iota16, offv])
                    hi = plsc.load_gather(bufs.at[slot], [iota16 + L, offv])
                    stag[p, pl.ds(e * K, L)] = lo
                    stag[p, pl.ds(e * K + L, L)] = hi
                for c in range(2 * L):
                    bidx[p, c // 8, pl.ds((c % 8) * L, L)] = (
                        jqp[c // 2] * K + (c % 2) * L + iota16)
                for q in range(4):
                    pltpu.async_copy(stag.at[p, pl.ds(q * 128, 128)],
                                     out_o.at[bidx.at[p, q]], ssem.at[p])
                return gcnt + 1

            return lax.fori_loop(0, (nq + L - 1) // L, group, gcnt)

        for s in range(NBUF):
            issue(tab, s, s)

        def step(it, gcnt):
            g0 = it * NBUF
            for jj in range(NBUF):
                g = g0 + jj

                def do(gcnt):
                    wait_win(tab, g, jj)
                    gcnt = extract_window(w0 + g * W, W, jj, gcnt)

                    @pl.when(g + NBUF < NWIN)
                    def _():
                        issue(tab, g + NBUF, jj)
                    return gcnt
                gcnt = lax.cond(g < NWIN, do, lambda c: c, gcnt)
            return gcnt

        gcnt = lax.fori_loop(0, (NWIN + NBUF - 1) // NBUF, step,
                             jnp.int32(0))

        @pl.when(rid == NW - 1)
        def _():
            pltpu.sync_copy(tab.at[:, pl.ds(TAIL0, W)], bufs.at[0])
            pltpu.sync_copy(tab.at[:, pl.ds(TAIL0 + W, W)], bufs.at[1])
        gcnt = lax.cond(
            rid == NW - 1,
            lambda c: extract_window(
                jnp.int32(TAIL0 + W), W, 1,
                extract_window(jnp.int32(TAIL0), W, 0, c)),
            lambda c: c, gcnt)

        @pl.when(gcnt >= 1)
        def _():
            drain_scatter(out_o, (gcnt - 1) % NBUF)

        @pl.when(gcnt >= 2)
        def _():
            drain_scatter(out_o, gcnt % NBUF)

        def bias_group(qb, _):
            n = nloc - qb * L
            jlast = lj_v[pl.ds(qb * L + n - 1, L)][0]
            ulast = lu_v[pl.ds(qb * L + n - 1, L)][0]
            uq = lu_v[pl.ds(qb * L, L)]
            jq = lj_v[pl.ds(qb * L, L)]
            bgi[0, pl.ds(0, L)] = jnp.where(iota16 < n, uq, ulast)
            pltpu.sync_copy(bias_hbm.at[bgi.at[0]], bgv)
            bgi[0, pl.ds(0, L)] = jnp.where(iota16 < n, jq, jlast)
            pltpu.sync_copy(bgv, bias_o.at[bgi.at[0]])
            return ()
        lax.fori_loop(0, (nloc + L - 1) // L, bias_group, ())

    one_table(t2u, u_hbm, ub_hbm, ue_o, ub_o)
    one_table(t2i, i_hbm, ib_hbm, ie_o, ib_o)


TCB = 2048


def _tc_body(ue, ie, ub, ib, u1, i1, ublk, iblk, out):
    f32 = jnp.float32
    iot = lax.broadcasted_iota(jnp.int32, (TCB, 64), 1) + TAIL1
    oh_u = (u1[...] == iot).astype(f32)
    oh_i = (i1[...] == iot).astype(f32)
    um = (u1[...] < TAIL1).astype(f32)
    im = (i1[...] < TAIL1).astype(f32)
    uev = ue[...] * um + jnp.dot(oh_u, ublk[...], preferred_element_type=f32)
    iev = ie[...] * im + jnp.dot(oh_i, iblk[...], preferred_element_type=f32)
    s = jnp.sum(uev * iev, axis=1)
    out[...] = s + ub[...] + ib[...]


def kernel(u, i, user_emb, item_emb, user_bias, item_bias):
    mesh = plsc.VectorSubcoreMesh(core_axis_name="c", subcore_axis_name="s",
                                  num_cores=NC, num_subcores=NS)
    f32 = jnp.float32
    run = pl.kernel(
        _sc_body,
        out_type=[
            jax.ShapeDtypeStruct((BATCH * K,), f32),
            jax.ShapeDtypeStruct((BATCH * K,), f32),
            jax.ShapeDtypeStruct((BATCH,), f32),
            jax.ShapeDtypeStruct((BATCH,), f32),
        ],
        mesh=mesh,
        compiler_params=pltpu.CompilerParams(needs_layout_passes=False,
                                             use_tc_tiling_on_sc=True),
        scratch_types=[
            pltpu.VMEM((64, 128), jnp.int32),
            pltpu.VMEM((BATCH + L,), jnp.int32),
            pltpu.VMEM((BATCH + L,), jnp.int32),
            pltpu.VMEM((NBUF, K, W), f32),
            pltpu.VMEM((BATCH + L,), jnp.int32),
            pltpu.VMEM((BATCH + L,), jnp.int32),
            pltpu.VMEM((NBUF, 512), f32),
            pltpu.VMEM((NBUF, 4, 128), jnp.int32),
            pltpu.VMEM((1, L), jnp.int32),
            pltpu.VMEM((L,), f32),
            pltpu.SemaphoreType.DMA((NBUF,)),
            pltpu.SemaphoreType.DMA((NBUF,)),
            pltpu.SemaphoreType.DMA,
        ],
    )
    u2 = u.astype(jnp.int32).reshape(128, 128)
    i2 = i.astype(jnp.int32).reshape(128, 128)
    ue_f, ie_f, ub_g, ib_g = run(u2, i2, user_emb.T, item_emb.T,
                                 user_bias.reshape(-1), item_bias.reshape(-1))

    u1 = u.astype(jnp.int32).reshape(BATCH, 1)
    i1 = i.astype(jnp.int32).reshape(BATCH, 1)
    dot = pl.pallas_call(
        _tc_body,
        grid=(BATCH // TCB,),
        in_specs=[
            pl.BlockSpec((TCB, K), lambda b: (b, 0)),
            pl.BlockSpec((TCB, K), lambda b: (b, 0)),
            pl.BlockSpec((TCB,), lambda b: (b,)),
            pl.BlockSpec((TCB,), lambda b: (b,)),
            pl.BlockSpec((TCB, 1), lambda b: (b, 0)),
            pl.BlockSpec((TCB, 1), lambda b: (b, 0)),
            pl.BlockSpec((64, K), lambda b: (0, 0)),
            pl.BlockSpec((64, K), lambda b: (0, 0)),
        ],
        out_specs=pl.BlockSpec((TCB,), lambda b: (b,)),
        out_shape=jax.ShapeDtypeStruct((BATCH,), f32),
    )
    return dot(ue_f.reshape(BATCH, K), ie_f.reshape(BATCH, K), ub_g, ib_g,
               u1, i1, user_emb[TAIL1:, :], item_emb[TAIL1:, :])


# speedup vs baseline: 1.0015x; 1.0015x over previous
import jax
import jax.numpy as jnp
from jax import lax
from jax.experimental import pallas as pl
from jax.experimental.pallas import tpu as pltpu
from jax.experimental.pallas import tpu_sc as plsc

NVOC = 1000000
K = 32
BATCH = 16384

NC = 2
NS = 16
NW = NC * NS
W = 256
NWIN = 122
RNG = NWIN * W
NBUF = 2
TAIL0 = RNG * NW
TAIL1 = TAIL0 + 2 * W
L = 16


def _sc_body(u_hbm, i_hbm, t2u, t2i, ub_hbm, ib_hbm,
             ue_o, ie_o, ub_o, ib_o,
             idx_v, lu_v, lj_v, bufs, su_v, sj_v, hist, starts, ptr,
             stag, bidx, bgi, bgv, sems, ssem, bsem):
    cid = lax.axis_index("c")
    sid = lax.axis_index("s")
    rid = cid * NS + sid
    w0 = rid * RNG
    iota16 = lax.iota(jnp.int32, L)

    def issue(tab, g, slot):
        for a in range(4):
            pltpu.async_copy(tab.at[pl.ds(a * 8, 8), pl.ds(w0 + g * W, W)],
                             bufs.at[slot, pl.ds(a * 8, 8), :], sems.at[slot])

    def wait_win(tab, g, slot):
        for a in range(4):
            pltpu.make_async_copy(
                tab.at[pl.ds(a * 8, 8), pl.ds(w0 + g * W, W)],
                bufs.at[slot, pl.ds(a * 8, 8), :], sems.at[slot]).wait()

    def drain_scatter(out_o, p):
        for q in range(4):
            pltpu.make_async_copy(stag.at[p, pl.ds(q * 128, 128)],
                                  out_o.at[bidx.at[p, q]],
                                  ssem.at[p]).wait()

    def one_table(tab, idx2d_hbm, bias_hbm, out_o, bias_o):
        def filt_half(h, pos):
            pltpu.sync_copy(idx2d_hbm.at[pl.ds(h * 64, 64), :], idx_v)

            def filt_row(r, pos):
                def sub(t, pos):
                    uv = idx_v[r, pl.ds(t * L, L)]
                    own = jnp.minimum(uv // RNG, NW - 1)
                    m = own == rid
                    jv = (h * 64 + r) * 128 + t * L + iota16
                    plsc.store_compressed(lu_v.at[pl.ds(pos, L)], uv, mask=m)
                    plsc.store_compressed(lj_v.at[pl.ds(pos, L)], jv, mask=m)
                    return pos + plsc.all_reduce_population_count(m)[0]
                for t in range(8):
                    pos = sub(t, pos)
                return pos
            return lax.fori_loop(0, 64, filt_row, pos)

        nloc = filt_half(0, jnp.int32(0))
        nloc = filt_half(1, nloc)

        zero16i = iota16 * 0
        m0 = iota16 == 0
        for c in range(9):
            hist[pl.ds(c * L, L)] = zero16i

        def hchunk(c, _):
            uv = lu_v[pl.ds(c * L, L)]
            m = c * L + iota16 < nloc
            wi = jnp.where(m, (uv - w0) // W, 126)
            for lane in range(L):
                plsc.addupdate_scatter(hist, [zero16i + wi[lane]],
                                       zero16i + 1, mask=m0)
            return ()
        lax.fori_loop(0, (nloc + L - 1) // L, hchunk, ())

        def prefix(c, base):
            v = hist[pl.ds(c * L, L)]
            cs = plsc.cumsum(v)
            starts[pl.ds(c * L, L)] = cs - v + base
            ptr[pl.ds(c * L, L)] = cs - v + base
            return base + cs[L - 1]
        lax.fori_loop(0, 9, prefix, jnp.int32(0))

        def place(e, _):
            u0 = lu_v[pl.ds(e, L)][0]
            j0 = lj_v[pl.ds(e, L)][0]
            wi = (u0 - w0) // W
            pos = ptr[pl.ds(wi, L)][0]
            plsc.store_scatter(su_v, [zero16i + pos], zero16i + u0, mask=m0)
            plsc.store_scatter(sj_v, [zero16i + pos], zero16i + j0, mask=m0)
            plsc.store_scatter(ptr, [zero16i + wi], zero16i + pos + 1,
                               mask=m0)
            return ()
        lax.fori_loop(0, nloc, place, ())

        def extract_window(g, v0, slot, gcnt):
            seg0 = starts[pl.ds(g, L)][0]
            seg1 = starts[pl.ds(g + 1, L)][0]

            def group(qb, gcnt):
                base = seg0 + qb * L
                n = seg1 - base
                p = gcnt % NBUF

                @pl.when(gcnt >= NBUF)
                def _():
                    drain_scatter(out_o, p)

                jlast = sj_v[pl.ds(base + n - 1, L)][0]
                ulast = su_v[pl.ds(base + n - 1, L)][0]
                jq = sj_v[pl.ds(base, L)]
                uq = su_v[pl.ds(base, L)]
                jqp = jnp.where(iota16 < n, jq, jlast)
                offs = jnp.where(iota16 < n, uq, ulast) - v0

                for e in range(L):
                    offv = iota16 * 0 + offs[e]
                    lo = plsc.load_gather(bufs.at[slot], [iota16, offv])
                    hi = plsc.load_gather(bufs.at[slot], [iota16 + L, offv])
                    stag[p, pl.ds(e * K, L)] = lo
                    stag[p, pl.ds(e * K + L, L)] = hi
                for c in range(2 * L):
                    bidx[p, c // 8, pl.ds((c % 8) * L, L)] = (
                        jqp[c // 2] * K + (c % 2) * L + iota16)
                for q in range(4):
                    pltpu.async_copy(stag.at[p, pl.ds(q * 128, 128)],
                                     out_o.at[bidx.at[p, q]], ssem.at[p])
                return gcnt + 1

            return lax.fori_loop(0, (seg1 - seg0 + L - 1) // L, group, gcnt)

        for s in range(NBUF):
            issue(tab, s, s)

        def step(it, gcnt):
            g0 = it * NBUF
            for jj in range(NBUF):
                g = g0 + jj

                def do(gcnt):
                    wait_win(tab, g, jj)
                    gcnt = extract_window(g, w0 + g * W, jj, gcnt)

                    @pl.when(g + NBUF < NWIN)
                    def _():
                        issue(tab, g + NBUF, jj)
                    return gcnt
                gcnt = lax.cond(g < NWIN, do, lambda c: c, gcnt)
            return gcnt

        gcnt = lax.fori_loop(0, (NWIN + NBUF - 1) // NBUF, step,
                             jnp.int32(0))

        @pl.when(rid == NW - 1)
        def _():
            pltpu.sync_copy(tab.at[:, pl.ds(TAIL0, W)], bufs.at[0])
            pltpu.sync_copy(tab.at[:, pl.ds(TAIL0 + W, W)], bufs.at[1])
        gcnt = lax.cond(
            rid == NW - 1,
            lambda c: extract_window(
                jnp.int32(NWIN + 1), jnp.int32(TAIL0 + W), 1,
                extract_window(jnp.int32(NWIN), jnp.int32(TAIL0), 0, c)),
            lambda c: c, gcnt)

        @pl.when(gcnt >= 1)
        def _():
            drain_scatter(out_o, (gcnt - 1) % NBUF)

        @pl.when(gcnt >= 2)
        def _():
            drain_scatter(out_o, gcnt % NBUF)

        def bias_group(qb, _):
            n = nloc - qb * L
            jlast = lj_v[pl.ds(qb * L + n - 1, L)][0]
            ulast = lu_v[pl.ds(qb * L + n - 1, L)][0]
            uq = lu_v[pl.ds(qb * L, L)]
            jq = lj_v[pl.ds(qb * L, L)]
            bgi[0, pl.ds(0, L)] = jnp.where(iota16 < n, uq, ulast)
            pltpu.sync_copy(bias_hbm.at[bgi.at[0]], bgv)
            bgi[0, pl.ds(0, L)] = jnp.where(iota16 < n, jq, jlast)
            pltpu.sync_copy(bgv, bias_o.at[bgi.at[0]])
            return ()
        lax.fori_loop(0, (nloc + L - 1) // L, bias_group, ())

    one_table(t2u, u_hbm, ub_hbm, ue_o, ub_o)
    one_table(t2i, i_hbm, ib_hbm, ie_o, ib_o)


TCB = 2048


def _tc_body(ue, ie, ub, ib, u1, i1, ublk, iblk, out):
    f32 = jnp.float32
    iot = lax.broadcasted_iota(jnp.int32, (TCB, 64), 1) + TAIL1
    oh_u = (u1[...] == iot).astype(f32)
    oh_i = (i1[...] == iot).astype(f32)
    um = (u1[...] < TAIL1).astype(f32)
    im = (i1[...] < TAIL1).astype(f32)
    uev = ue[...] * um + jnp.dot(oh_u, ublk[...], preferred_element_type=f32)
    iev = ie[...] * im + jnp.dot(oh_i, iblk[...], preferred_element_type=f32)
    s = jnp.sum(uev * iev, axis=1)
    out[...] = s + ub[...] + ib[...]


def kernel(u, i, user_emb, item_emb, user_bias, item_bias):
    mesh = plsc.VectorSubcoreMesh(core_axis_name="c", subcore_axis_name="s",
                                  num_cores=NC, num_subcores=NS)
    f32 = jnp.float32
    run = pl.kernel(
        _sc_body,
        out_type=[
            jax.ShapeDtypeStruct((BATCH * K,), f32),
            jax.ShapeDtypeStruct((BATCH * K,), f32),
            jax.ShapeDtypeStruct((BATCH,), f32),
            jax.ShapeDtypeStruct((BATCH,), f32),
        ],
        mesh=mesh,
        compiler_params=pltpu.CompilerParams(needs_layout_passes=False,
                                             use_tc_tiling_on_sc=True),
        scratch_types=[
            pltpu.VMEM((64, 128), jnp.int32),
            pltpu.VMEM((BATCH + L,), jnp.int32),
            pltpu.VMEM((BATCH + L,), jnp.int32),
            pltpu.VMEM((NBUF, K, W), f32),
            pltpu.VMEM((BATCH + L,), jnp.int32),
            pltpu.VMEM((BATCH + L,), jnp.int32),
            pltpu.VMEM((144,), jnp.int32),
            pltpu.VMEM((144,), jnp.int32),
            pltpu.VMEM((144,), jnp.int32),
            pltpu.VMEM((NBUF, 512), f32),
            pltpu.VMEM((NBUF, 4, 128), jnp.int32),
            pltpu.VMEM((1, L), jnp.int32),
            pltpu.VMEM((L,), f32),
            pltpu.SemaphoreType.DMA((NBUF,)),
            pltpu.SemaphoreType.DMA((NBUF,)),
            pltpu.SemaphoreType.DMA,
        ],
    )
    u2 = u.astype(jnp.int32).reshape(128, 128)
    i2 = i.astype(jnp.int32).reshape(128, 128)
    ue_f, ie_f, ub_g, ib_g = run(u2, i2, user_emb.T, item_emb.T,
                                 user_bias.reshape(-1), item_bias.reshape(-1))

    u1 = u.astype(jnp.int32).reshape(BATCH, 1)
    i1 = i.astype(jnp.int32).reshape(BATCH, 1)
    dot = pl.pallas_call(
        _tc_body,
        grid=(BATCH // TCB,),
        in_specs=[
            pl.BlockSpec((TCB, K), lambda b: (b, 0)),
            pl.BlockSpec((TCB, K), lambda b: (b, 0)),
            pl.BlockSpec((TCB,), lambda b: (b,)),
            pl.BlockSpec((TCB,), lambda b: (b,)),
            pl.BlockSpec((TCB, 1), lambda b: (b, 0)),
            pl.BlockSpec((TCB, 1), lambda b: (b, 0)),
            pl.BlockSpec((64, K), lambda b: (0, 0)),
            pl.BlockSpec((64, K), lambda b: (0, 0)),
        ],
        out_specs=pl.BlockSpec((TCB,), lambda b: (b,)),
        out_shape=jax.ShapeDtypeStruct((BATCH,), f32),
    )
    return dot(ue_f.reshape(BATCH, K), ie_f.reshape(BATCH, K), ub_g, ib_g,
               u1, i1, user_emb[TAIL1:, :], item_emb[TAIL1:, :])


# speedup vs baseline: 11.7162x; 11.6991x over previous
import functools

import jax
import jax.numpy as jnp
from jax import lax
from jax.experimental import pallas as pl
from jax.experimental.pallas import tpu as pltpu
from jax.experimental.pallas import tpu_sc as plsc

N_USERS = 1000000
N_ITEMS = 1000000
K = 32
BATCH = 16384

NC = 2
NS = 16
NW = NC * NS
B_PER_W = BATCH // NW
CHUNK = 128
NCHUNK = B_PER_W // CHUNK
L = 16


def _sc_body(u_hbm, i_hbm, ue_hbm, ie_hbm, ub_hbm, ib_hbm, out_hbm,
             idx_u, idx_i, rows_u, rows_i, bias_u, bias_i, out_v, sem):
    wid = lax.axis_index("s") * NC + lax.axis_index("c")
    pltpu.sync_copy(u_hbm.at[pl.ds(wid * NCHUNK, NCHUNK)], idx_u)
    pltpu.sync_copy(i_hbm.at[pl.ds(wid * NCHUNK, NCHUNK)], idx_i)

    copies = []
    for j in range(NCHUNK):
        rsl = pl.ds(j * CHUNK, CHUNK)
        copies.append(pltpu.async_copy(ue_hbm.at[idx_u.at[j]], rows_u.at[rsl], sem))
        copies.append(pltpu.async_copy(ie_hbm.at[idx_i.at[j]], rows_i.at[rsl], sem))
        copies.append(pltpu.async_copy(ub_hbm.at[idx_u.at[j]], bias_u.at[rsl], sem))
        copies.append(pltpu.async_copy(ib_hbm.at[idx_i.at[j]], bias_i.at[rsl], sem))
    for c in copies:
        c.wait()

    def group(g, _):
        rid = g * L + lax.iota(jnp.int32, L)
        acc = bias_u[pl.ds(g * L, L)] + bias_i[pl.ds(g * L, L)]
        for k in range(K):
            kv = jnp.full((L,), k, jnp.int32)
            uv = plsc.load_gather(rows_u, [rid, kv])
            iv = plsc.load_gather(rows_i, [rid, kv])
            acc = acc + uv * iv
        out_v[pl.ds(g * L, L)] = acc
        return ()

    lax.fori_loop(0, B_PER_W // L, group, (), unroll=1)

    pltpu.sync_copy(out_v, out_hbm.at[pl.ds(wid * B_PER_W, B_PER_W)])


@functools.partial(jax.jit, static_argnames=())
def kernel(u, i, user_emb, item_emb, user_bias, item_bias):
    mesh = plsc.VectorSubcoreMesh(core_axis_name="c", subcore_axis_name="s",
                                  num_cores=NC, num_subcores=NS)
    run = pl.kernel(
        _sc_body,
        out_type=jax.ShapeDtypeStruct((BATCH,), jnp.float32),
        mesh=mesh,
        compiler_params=pltpu.CompilerParams(needs_layout_passes=False,
                                             use_tc_tiling_on_sc=False),
        scratch_types=[
            pltpu.VMEM((NCHUNK, CHUNK), jnp.int32),
            pltpu.VMEM((NCHUNK, CHUNK), jnp.int32),
            pltpu.VMEM((B_PER_W, K), jnp.float32),
            pltpu.VMEM((B_PER_W, K), jnp.float32),
            pltpu.VMEM((B_PER_W,), jnp.float32),
            pltpu.VMEM((B_PER_W,), jnp.float32),
            pltpu.VMEM((B_PER_W,), jnp.float32),
            pltpu.SemaphoreType.DMA,
        ],
    )
    u2 = u.astype(jnp.int32).reshape(NW * NCHUNK, CHUNK)
    i2 = i.astype(jnp.int32).reshape(NW * NCHUNK, CHUNK)
    return run(u2, i2, user_emb, item_emb,
               user_bias.reshape(-1), item_bias.reshape(-1))
